# baseline (speedup 1.0000x reference)
import jax
import jax.numpy as jnp
from jax import lax
from jax.experimental import pallas as pl
from jax.experimental.pallas import tpu as pltpu
from jax.experimental.pallas import tpu_sc as plsc

B, P = 4, 2048
N = B * P
SHAPE = 256
GEO = 4
E = 8
LAT = 1024
K = 128
NBLK = (N + E * (K - 1) + K - 1) // K
NPAD = NBLK * K
TW = 384

NEG = -3e38

NW = 32
RPW = N // NW
NCH = 2
CH = RPW // NCH


def _sc_scatter_body(cg_hbm, dest_hbm, out_hbm, idx_v, rows_v, sem_i, sem_o):
    wid = lax.axis_index("s") * 2 + lax.axis_index("c")
    pltpu.sync_copy(dest_hbm.at[wid], idx_v)
    loads = [pltpu.async_copy(cg_hbm.at[wid * NCH + j], rows_v.at[j], sem_i)
             for j in range(NCH)]
    stores = []
    for j in range(NCH):
        loads[j].wait()
        stores.append(
            pltpu.async_copy(rows_v.at[j], out_hbm.at[idx_v.at[j]], sem_o))
    for c in stores:
        c.wait()


def _sc_scatter(cg3, dest3):
    mesh = plsc.VectorSubcoreMesh(core_axis_name="c", subcore_axis_name="s")
    return pl.kernel(
        _sc_scatter_body,
        mesh=mesh,
        out_type=jax.ShapeDtypeStruct((NPAD, TW), jnp.float32),
        scratch_types=[
            pltpu.VMEM((NCH, CH), jnp.int32),
            pltpu.VMEM((NCH, CH, TW), jnp.float32),
            pltpu.SemaphoreType.DMA,
            pltpu.SemaphoreType.DMA,
        ],
    )(cg3, dest3)


def _tc_body(be_ref, st_ref, en_ref, ts_ref,
             W1_ref, b1_ref, W2_ref, b2_ref, W3_ref, b3_ref,
             C1e_ref, C1g_ref, CB1_ref, C2_ref, CB2_ref, C3_ref, CB3_ref,
             out_ref):
    i = pl.program_id(0)
    bf16, f32 = jnp.bfloat16, jnp.float32
    codes = ts_ref[:, :SHAPE].astype(bf16)
    bg = ts_ref[:, SHAPE:].astype(bf16)

    h = jnp.dot(codes, W1_ref[0], preferred_element_type=f32) + b1_ref[0]
    h = jnp.maximum(h, 0.0).astype(bf16)
    h = jnp.dot(h, W2_ref[0], preferred_element_type=f32) + b2_ref[0]
    h = jnp.maximum(h, 0.0).astype(bf16)
    enc = jnp.dot(h, W3_ref[0], preferred_element_type=f32) + b3_ref[0]

    t = (jnp.dot(enc.astype(bf16), C1e_ref[...], preferred_element_type=f32)
         + jnp.dot(bg, C1g_ref[...], preferred_element_type=f32)
         + CB1_ref[...])
    t = jnp.maximum(t, 0.0).astype(bf16)
    t = jnp.dot(t, C2_ref[...], preferred_element_type=f32) + CB2_ref[...]
    t = jnp.maximum(t, 0.0).astype(bf16)
    t = jnp.dot(t, C3_ref[...], preferred_element_type=f32) + CB3_ref[...]

    @pl.when(i == 0)
    def _():
        out_ref[...] = jnp.full((B, LAT), NEG, dtype=jnp.float32)

    iota = lax.broadcasted_iota(jnp.int32, (K, 1), 0)
    for b in range(B):
        stb = st_ref[i, b]
        enb = en_ref[i, b]

        @pl.when(enb > stb)
        def _(stb=stb, enb=enb, b=b):
            mb = jnp.max(jnp.where((iota >= stb) & (iota < enb), t, NEG),
                         axis=0)
            out_ref[b, :] = jnp.maximum(out_ref[b, :], mb)


def _tc_call(be, st, en, ts, W1, b1, W2, b2, W3, b3,
             C1e, C1g, CB1, C2, CB2, C3, CB3):
    grid_spec = pltpu.PrefetchScalarGridSpec(
        num_scalar_prefetch=3,
        grid=(NBLK,),
        in_specs=[
            pl.BlockSpec((K, TW), lambda i, be, st, en: (i, 0)),
            pl.BlockSpec((1, SHAPE, 512), lambda i, be, st, en: (be[i], 0, 0)),
            pl.BlockSpec((1, 1, 512), lambda i, be, st, en: (be[i], 0, 0)),
            pl.BlockSpec((1, 512, 512), lambda i, be, st, en: (be[i], 0, 0)),
            pl.BlockSpec((1, 1, 512), lambda i, be, st, en: (be[i], 0, 0)),
            pl.BlockSpec((1, 512, SHAPE), lambda i, be, st, en: (be[i], 0, 0)),
            pl.BlockSpec((1, 1, SHAPE), lambda i, be, st, en: (be[i], 0, 0)),
            pl.BlockSpec((SHAPE, 512), lambda i, be, st, en: (0, 0)),
            pl.BlockSpec((K, 512), lambda i, be, st, en: (0, 0)),
            pl.BlockSpec((1, 512), lambda i, be, st, en: (0, 0)),
            pl.BlockSpec((512, 1024), lambda i, be, st, en: (0, 0)),
            pl.BlockSpec((1, 1024), lambda i, be, st, en: (0, 0)),
            pl.BlockSpec((1024, LAT), lambda i, be, st, en: (0, 0)),
            pl.BlockSpec((1, LAT), lambda i, be, st, en: (0, 0)),
        ],
        out_specs=pl.BlockSpec((B, LAT), lambda i, be, st, en: (0, 0)),
    )
    return pl.pallas_call(
        _tc_body,
        grid_spec=grid_spec,
        out_shape=jax.ShapeDtypeStruct((B, LAT), jnp.float32),
    )(be, st, en, ts, W1, b1, W2, b2, W3, b3,
      C1e, C1g, CB1, C2, CB2, C3, CB3)


def kernel(x, cats, W1, b1, W2, b2, W3, b3, CW1, CB1, CW2, CB2, CW3, CB3):
    xt = jnp.transpose(x, (0, 2, 1)).reshape(N, GEO + SHAPE)
    cg = jnp.concatenate(
        [xt[:, GEO:], xt[:, :GEO],
         jnp.zeros((N, TW - SHAPE - GEO), jnp.float32)], axis=1)

    cf = cats.reshape(-1).astype(jnp.int32)
    oh = (cf[:, None] == jnp.arange(E, dtype=jnp.int32)).astype(jnp.int32)
    cum = jnp.cumsum(oh, axis=0)
    rank = jnp.sum((cum - oh) * oh, axis=1)
    counts = cum[-1]
    padded = ((counts + K - 1) // K) * K
    ends = jnp.cumsum(padded)
    off = ends - padded
    dest = jnp.sum(oh * off[None, :], axis=1) + rank

    ceb = jnp.sum(oh.reshape(B, P, E), axis=1)
    seb = jnp.cumsum(ceb, axis=0) - ceb

    bstart = jnp.arange(NBLK, dtype=jnp.int32) * K
    be = jnp.minimum(
        jnp.sum((bstart[:, None] >= ends[None, :]).astype(jnp.int32), axis=1),
        E - 1).astype(jnp.int32)
    beoh = (be[:, None] == jnp.arange(E, dtype=jnp.int32)).astype(jnp.int32)
    r0 = bstart - jnp.sum(beoh * off[None, :], axis=1)
    s_sel = jnp.sum(beoh[:, None, :] * seb[None, :, :], axis=2)
    c_sel = jnp.sum(beoh[:, None, :] * ceb[None, :, :], axis=2)
    st = jnp.clip(s_sel - r0[:, None], 0, K).astype(jnp.int32)
    en = jnp.clip(s_sel + c_sel - r0[:, None], 0, K).astype(jnp.int32)

    ts = _sc_scatter(cg.reshape(NW * NCH, CH, TW),
                     dest.reshape(NW, NCH, CH))

    bf16 = jnp.bfloat16
    C1g = jnp.concatenate(
        [CW1[:GEO], jnp.zeros((K - GEO, 512), jnp.float32)], axis=0)
    out = _tc_call(
        be, st, en, ts,
        W1.astype(bf16), b1.reshape(E, 1, 512),
        W2.astype(bf16), b2.reshape(E, 1, 512),
        W3.astype(bf16), b3.reshape(E, 1, SHAPE),
        CW1[GEO:].astype(bf16), C1g.astype(bf16), CB1.reshape(1, 512),
        CW2.astype(bf16), CB2.reshape(1, 1024),
        CW3.astype(bf16), CB3.reshape(1, LAT))
    return out


# speedup vs baseline: 1.0437x; 1.0437x over previous
import jax
import jax.numpy as jnp
from jax import lax
from jax.experimental import pallas as pl
from jax.experimental.pallas import tpu as pltpu
from jax.experimental.pallas import tpu_sc as plsc

B, P = 4, 2048
N = B * P
SHAPE = 256
GEO = 4
E = 8
LAT = 1024
K = 128
NBLK = (N + E * (K - 1) + K - 1) // K
NPAD = NBLK * K
TW = 384

NEG = -3e38

NW = 32
RPW = N // NW
NCH = 2
CH = RPW // NCH


def _sc_scatter_body(cg_hbm, dest_hbm, out_hbm, idx_v, rows_v, sem_i, sem_o):
    wid = lax.axis_index("s") * 2 + lax.axis_index("c")
    pltpu.sync_copy(dest_hbm.at[wid], idx_v)
    loads = [pltpu.async_copy(cg_hbm.at[wid * NCH + j], rows_v.at[j], sem_i)
             for j in range(NCH)]
    stores = []
    for j in range(NCH):
        loads[j].wait()
        stores.append(
            pltpu.async_copy(rows_v.at[j], out_hbm.at[idx_v.at[j]], sem_o))
    for c in stores:
        c.wait()


def _sc_scatter(cg3, dest3):
    mesh = plsc.VectorSubcoreMesh(core_axis_name="c", subcore_axis_name="s")
    return pl.kernel(
        _sc_scatter_body,
        mesh=mesh,
        out_type=jax.ShapeDtypeStruct((NPAD, TW), jnp.float32),
        scratch_types=[
            pltpu.VMEM((NCH, CH), jnp.int32),
            pltpu.VMEM((NCH, CH, TW), jnp.float32),
            pltpu.SemaphoreType.DMA,
            pltpu.SemaphoreType.DMA,
        ],
    )(cg3, dest3)


def _tc_body(be_ref, st_ref, en_ref, ts_ref,
             W1_ref, b12_ref, W2_ref, W3C1_ref, b3C1_ref,
             C1g_ref, C2_ref, CB2_ref, C3_ref, CB3_ref,
             out_ref):
    i = pl.program_id(0)
    e = be_ref[i]
    bf16, f32 = jnp.bfloat16, jnp.float32
    codes = ts_ref[:, :SHAPE].astype(bf16)
    bg = ts_ref[:, SHAPE:].astype(bf16)

    h = jnp.dot(codes, W1_ref[e], preferred_element_type=f32) + b12_ref[e, :1]
    h = jnp.maximum(h, 0.0).astype(bf16)
    h = jnp.dot(h, W2_ref[e], preferred_element_type=f32) + b12_ref[e, 1:]
    h = jnp.maximum(h, 0.0).astype(bf16)

    t = (jnp.dot(h, W3C1_ref[e], preferred_element_type=f32)
         + jnp.dot(bg, C1g_ref[...], preferred_element_type=f32)
         + b3C1_ref[e])
    t = jnp.maximum(t, 0.0).astype(bf16)
    t = jnp.dot(t, C2_ref[...], preferred_element_type=f32) + CB2_ref[...]
    t = jnp.maximum(t, 0.0).astype(bf16)
    t = jnp.dot(t, C3_ref[...], preferred_element_type=f32) + CB3_ref[...]

    iota = lax.broadcasted_iota(jnp.int32, (K, 1), 0)
    mx = [jnp.max(jnp.where((iota >= st_ref[i, b]) & (iota < en_ref[i, b]),
                            t, NEG), axis=0)
          for b in range(B)]
    res = jnp.stack(mx)

    @pl.when(i == 0)
    def _():
        out_ref[...] = jnp.full((B, LAT), NEG, dtype=jnp.float32)

    out_ref[...] = jnp.maximum(out_ref[...], res)


def _tc_call(be, st, en, ts, W1, b12, W2, W3C1, b3C1, C1g, C2, CB2, C3, CB3):
    full = lambda *shape: pl.BlockSpec(shape, lambda i, be, st, en:
                                       (0,) * len(shape))
    grid_spec = pltpu.PrefetchScalarGridSpec(
        num_scalar_prefetch=3,
        grid=(NBLK,),
        in_specs=[
            pl.BlockSpec((K, TW), lambda i, be, st, en: (i, 0)),
            full(E, SHAPE, 512),
            full(E, 2, 512),
            full(E, 512, 512),
            full(E, 512, 512),
            full(E, 1, 512),
            full(K, 512),
            full(512, 1024),
            full(1, 1024),
            full(1024, LAT),
            full(1, LAT),
        ],
        out_specs=pl.BlockSpec((B, LAT), lambda i, be, st, en: (0, 0)),
    )
    return pl.pallas_call(
        _tc_body,
        grid_spec=grid_spec,
        out_shape=jax.ShapeDtypeStruct((B, LAT), jnp.float32),
    )(be, st, en, ts, W1, b12, W2, W3C1, b3C1, C1g, C2, CB2, C3, CB3)


def kernel(x, cats, W1, b1, W2, b2, W3, b3, CW1, CB1, CW2, CB2, CW3, CB3):
    xt = jnp.transpose(x, (0, 2, 1)).reshape(N, GEO + SHAPE)
    cg = jnp.concatenate(
        [xt[:, GEO:], xt[:, :GEO],
         jnp.zeros((N, TW - SHAPE - GEO), jnp.float32)], axis=1)

    cf = cats.reshape(-1).astype(jnp.int32)
    oh = (cf[:, None] == jnp.arange(E, dtype=jnp.int32)).astype(jnp.int32)
    cum = jnp.cumsum(oh, axis=0)
    rank = jnp.sum((cum - oh) * oh, axis=1)
    counts = cum[-1]
    padded = ((counts + K - 1) // K) * K
    ends = jnp.cumsum(padded)
    off = ends - padded
    dest = jnp.sum(oh * off[None, :], axis=1) + rank

    ceb = jnp.sum(oh.reshape(B, P, E), axis=1)
    seb = jnp.cumsum(ceb, axis=0) - ceb

    bstart = jnp.arange(NBLK, dtype=jnp.int32) * K
    be = jnp.minimum(
        jnp.sum((bstart[:, None] >= ends[None, :]).astype(jnp.int32), axis=1),
        E - 1).astype(jnp.int32)
    beoh = (be[:, None] == jnp.arange(E, dtype=jnp.int32)).astype(jnp.int32)
    r0 = bstart - jnp.sum(beoh * off[None, :], axis=1)
    s_sel = jnp.sum(beoh[:, None, :] * seb[None, :, :], axis=2)
    c_sel = jnp.sum(beoh[:, None, :] * ceb[None, :, :], axis=2)
    st = jnp.clip(s_sel - r0[:, None], 0, K).astype(jnp.int32)
    en = jnp.clip(s_sel + c_sel - r0[:, None], 0, K).astype(jnp.int32)

    ts = _sc_scatter(cg.reshape(NW * NCH, CH, TW),
                     dest.reshape(NW, NCH, CH))

    bf16 = jnp.bfloat16
    C1e = CW1[GEO:]
    W3C1 = jnp.einsum('eij,jk->eik', W3, C1e)
    b3C1 = (b3 @ C1e + CB1[None, :]).reshape(E, 1, 512)
    C1g = jnp.concatenate(
        [CW1[:GEO], jnp.zeros((K - GEO, 512), jnp.float32)], axis=0)
    b12 = jnp.stack([b1, b2], axis=1)
    out = _tc_call(
        be, st, en, ts,
        W1.astype(bf16), b12, W2.astype(bf16),
        W3C1.astype(bf16), b3C1,
        C1g.astype(bf16), CW2.astype(bf16), CB2.reshape(1, 1024),
        CW3.astype(bf16), CB3.reshape(1, LAT))
    return out


# speedup vs baseline: 1.0631x; 1.0185x over previous
import jax
import jax.numpy as jnp
from jax import lax
from jax.experimental import pallas as pl
from jax.experimental.pallas import tpu as pltpu
from jax.experimental.pallas import tpu_sc as plsc

B, P = 4, 2048
N = B * P
SHAPE = 256
GEO = 4
E = 8
LAT = 1024
K = 128
NBLK = (N + E * (K - 1) + K - 1) // K
NPAD = NBLK * K
TW = 384

NEG = -3e38

NW = 32
RPW = N // NW
NCH = 2
CH = RPW // NCH


def _sc_scatter_body(cg_hbm, dest_hbm, out_hbm, idx_v, rows_v, sem_i, sem_o):
    wid = lax.axis_index("s") * 2 + lax.axis_index("c")
    pltpu.sync_copy(dest_hbm.at[wid], idx_v)
    loads = [pltpu.async_copy(cg_hbm.at[pl.ds(wid * RPW + j * CH, CH)],
                              rows_v.at[j], sem_i)
             for j in range(NCH)]
    stores = []
    for j in range(NCH):
        loads[j].wait()
        stores.append(
            pltpu.async_copy(rows_v.at[j], out_hbm.at[idx_v.at[j]], sem_o))
    for c in stores:
        c.wait()


def _sc_scatter(cg, dest3):
    mesh = plsc.VectorSubcoreMesh(core_axis_name="c", subcore_axis_name="s")
    return pl.kernel(
        _sc_scatter_body,
        mesh=mesh,
        out_type=jax.ShapeDtypeStruct((NPAD, TW), jnp.float32),
        scratch_types=[
            pltpu.VMEM((NCH, CH), jnp.int32),
            pltpu.VMEM((NCH, CH, TW), jnp.float32),
            pltpu.SemaphoreType.DMA,
            pltpu.SemaphoreType.DMA,
        ],
    )(cg, dest3)


def _tc_body(be_ref, st_ref, en_ref, ts_ref,
             W1_ref, b12_ref, W2_ref, W3C1_ref, b3C1_ref,
             C1g_ref, C2_ref, CB2_ref, C3_ref, CB3_ref,
             out_ref):
    i = pl.program_id(0)
    e = be_ref[i]
    bf16, f32 = jnp.bfloat16, jnp.float32
    codes = ts_ref[:, :SHAPE].astype(bf16)
    bg = ts_ref[:, SHAPE:].astype(bf16)

    h = jnp.dot(codes, W1_ref[e], preferred_element_type=f32) + b12_ref[e, :1]
    h = jnp.maximum(h, 0.0).astype(bf16)
    h = jnp.dot(h, W2_ref[e], preferred_element_type=f32) + b12_ref[e, 1:]
    h = jnp.maximum(h, 0.0).astype(bf16)

    t = (jnp.dot(h, W3C1_ref[e], preferred_element_type=f32)
         + jnp.dot(bg, C1g_ref[...], preferred_element_type=f32)
         + b3C1_ref[e])
    t = jnp.maximum(t, 0.0).astype(bf16)
    t = jnp.dot(t, C2_ref[...], preferred_element_type=f32) + CB2_ref[...]
    t = jnp.maximum(t, 0.0).astype(bf16)
    t = jnp.dot(t, C3_ref[...], preferred_element_type=f32) + CB3_ref[...]

    iota = lax.broadcasted_iota(jnp.int32, (K, 1), 0)
    mx = [jnp.max(jnp.where((iota >= st_ref[i, b]) & (iota < en_ref[i, b]),
                            t, NEG), axis=0)
          for b in range(B)]
    res = jnp.stack(mx)

    @pl.when(i == 0)
    def _():
        out_ref[...] = jnp.full((B, LAT), NEG, dtype=jnp.float32)

    out_ref[...] = jnp.maximum(out_ref[...], res)


def _tc_call(be, st, en, ts, W1, b12, W2, W3C1, b3C1, C1g, C2, CB2, C3, CB3):
    full = lambda *shape: pl.BlockSpec(shape, lambda i, be, st, en:
                                       (0,) * len(shape))
    grid_spec = pltpu.PrefetchScalarGridSpec(
        num_scalar_prefetch=3,
        grid=(NBLK,),
        in_specs=[
            pl.BlockSpec((K, TW), lambda i, be, st, en: (i, 0)),
            full(E, SHAPE, 512),
            full(E, 2, 512),
            full(E, 512, 512),
            full(E, 512, 512),
            full(E, 1, 512),
            full(K, 512),
            full(512, 1024),
            full(1, 1024),
            full(1024, LAT),
            full(1, LAT),
        ],
        out_specs=pl.BlockSpec((B, LAT), lambda i, be, st, en: (0, 0)),
    )
    return pl.pallas_call(
        _tc_body,
        grid_spec=grid_spec,
        out_shape=jax.ShapeDtypeStruct((B, LAT), jnp.float32),
    )(be, st, en, ts, W1, b12, W2, W3C1, b3C1, C1g, C2, CB2, C3, CB3)


def kernel(x, cats, W1, b1, W2, b2, W3, b3, CW1, CB1, CW2, CB2, CW3, CB3):
    cg = jnp.concatenate(
        [jnp.transpose(x[:, GEO:, :], (0, 2, 1)).reshape(N, SHAPE),
         jnp.transpose(x[:, :GEO, :], (0, 2, 1)).reshape(N, GEO),
         jnp.zeros((N, TW - SHAPE - GEO), jnp.float32)], axis=1)

    cf = cats.reshape(-1).astype(jnp.int32)
    oh = (cf[:, None] == jnp.arange(E, dtype=jnp.int32)).astype(jnp.int32)
    cum = jnp.cumsum(oh, axis=0)
    rank = jnp.sum((cum - oh) * oh, axis=1)
    counts = cum[-1]
    padded = ((counts + K - 1) // K) * K
    ends = jnp.cumsum(padded)
    off = ends - padded
    dest = jnp.sum(oh * off[None, :], axis=1) + rank

    ceb = jnp.sum(oh.reshape(B, P, E), axis=1)
    seb = jnp.cumsum(ceb, axis=0) - ceb

    bstart = jnp.arange(NBLK, dtype=jnp.int32) * K
    be = jnp.minimum(
        jnp.sum((bstart[:, None] >= ends[None, :]).astype(jnp.int32), axis=1),
        E - 1).astype(jnp.int32)
    beoh = (be[:, None] == jnp.arange(E, dtype=jnp.int32)).astype(jnp.int32)
    r0 = bstart - jnp.sum(beoh * off[None, :], axis=1)
    s_sel = jnp.sum(beoh[:, None, :] * seb[None, :, :], axis=2)
    c_sel = jnp.sum(beoh[:, None, :] * ceb[None, :, :], axis=2)
    st = jnp.clip(s_sel - r0[:, None], 0, K).astype(jnp.int32)
    en = jnp.clip(s_sel + c_sel - r0[:, None], 0, K).astype(jnp.int32)

    ts = _sc_scatter(cg, dest.reshape(NW, NCH, CH))

    bf16 = jnp.bfloat16
    C1e = CW1[GEO:]
    W3C1 = jnp.einsum('eij,jk->eik', W3, C1e)
    b3C1 = (b3 @ C1e + CB1[None, :]).reshape(E, 1, 512)
    C1g = jnp.concatenate(
        [CW1[:GEO], jnp.zeros((K - GEO, 512), jnp.float32)], axis=0)
    b12 = jnp.stack([b1, b2], axis=1)
    out = _tc_call(
        be, st, en, ts,
        W1.astype(bf16), b12, W2.astype(bf16),
        W3C1.astype(bf16), b3C1,
        C1g.astype(bf16), CW2.astype(bf16), CB2.reshape(1, 1024),
        CW3.astype(bf16), CB3.reshape(1, LAT))
    return out


# speedup vs baseline: 1.0806x; 1.0165x over previous
import jax
import jax.numpy as jnp
from jax import lax
from jax.experimental import pallas as pl
from jax.experimental.pallas import tpu as pltpu
from jax.experimental.pallas import tpu_sc as plsc

B, P = 4, 2048
N = B * P
SHAPE = 256
GEO = 4
E = 8
LAT = 1024
K = 128
NBLK = (N + E * (K - 1) + K - 1) // K
NPAD = NBLK * K
TW = 384

NEG = -3e38

NW = 32
RPW = N // NW
NCH = 2
CH = RPW // NCH


def _sc_scatter_body(cg_hbm, dest_hbm, out_hbm, idx_v, rows_v, sem_i, sem_o):
    wid = lax.axis_index("s") * 2 + lax.axis_index("c")
    pltpu.sync_copy(dest_hbm.at[wid], idx_v)
    loads = [pltpu.async_copy(cg_hbm.at[pl.ds(wid * RPW + j * CH, CH)],
                              rows_v.at[j], sem_i)
             for j in range(NCH)]
    stores = []
    for j in range(NCH):
        loads[j].wait()
        stores.append(
            pltpu.async_copy(rows_v.at[j], out_hbm.at[idx_v.at[j]], sem_o))
    for c in stores:
        c.wait()


def _sc_scatter(cg, dest3):
    mesh = plsc.VectorSubcoreMesh(core_axis_name="c", subcore_axis_name="s")
    return pl.kernel(
        _sc_scatter_body,
        mesh=mesh,
        out_type=jax.ShapeDtypeStruct((NPAD, TW), jnp.float32),
        scratch_types=[
            pltpu.VMEM((NCH, CH), jnp.int32),
            pltpu.VMEM((NCH, CH, TW), jnp.float32),
            pltpu.SemaphoreType.DMA,
            pltpu.SemaphoreType.DMA,
        ],
    )(cg, dest3)


def _tc_body(be_ref, st_ref, en_ref, ts_ref,
             W1_ref, b12_ref, W2_ref, W3C1_ref, b3C1_ref,
             C1g_ref, C2_ref, CB2_ref, C3_ref, CB3_ref,
             out_ref):
    i = pl.program_id(0)
    e = be_ref[i]
    bf16, f32 = jnp.bfloat16, jnp.float32

    @pl.when(i == 0)
    def _():
        out_ref[...] = jnp.full((B, LAT), NEG, dtype=jnp.float32)

    used = ((en_ref[i, 0] > st_ref[i, 0]) | (en_ref[i, 1] > st_ref[i, 1])
            | (en_ref[i, 2] > st_ref[i, 2]) | (en_ref[i, 3] > st_ref[i, 3]))

    @pl.when(used)
    def _():
        codes = ts_ref[:, :SHAPE].astype(bf16)
        bg = ts_ref[:, SHAPE:].astype(bf16)

        h = jnp.dot(codes, W1_ref[e],
                    preferred_element_type=f32) + b12_ref[e, :1]
        h = jnp.maximum(h, 0.0).astype(bf16)
        h = jnp.dot(h, W2_ref[e], preferred_element_type=f32) + b12_ref[e, 1:]
        h = jnp.maximum(h, 0.0).astype(bf16)

        t = (jnp.dot(h, W3C1_ref[e], preferred_element_type=f32)
             + jnp.dot(bg, C1g_ref[...], preferred_element_type=f32)
             + b3C1_ref[e])
        t = jnp.maximum(t, 0.0).astype(bf16)
        t = jnp.dot(t, C2_ref[...], preferred_element_type=f32) + CB2_ref[...]
        t = jnp.maximum(t, 0.0).astype(bf16)
        t = jnp.dot(t, C3_ref[...], preferred_element_type=f32) + CB3_ref[...]

        iota = lax.broadcasted_iota(jnp.int32, (K, 1), 0)
        mx = [jnp.max(jnp.where((iota >= st_ref[i, b])
                                & (iota < en_ref[i, b]), t, NEG), axis=0)
              for b in range(B)]
        res = jnp.stack(mx)
        out_ref[...] = jnp.maximum(out_ref[...], res)


def _tc_call(be, st, en, ts, W1, b12, W2, W3C1, b3C1, C1g, C2, CB2, C3, CB3):
    full = lambda *shape: pl.BlockSpec(shape, lambda i, be, st, en:
                                       (0,) * len(shape))
    grid_spec = pltpu.PrefetchScalarGridSpec(
        num_scalar_prefetch=3,
        grid=(NBLK,),
        in_specs=[
            pl.BlockSpec((K, TW), lambda i, be, st, en: (i, 0)),
            full(E, SHAPE, 512),
            full(E, 2, 512),
            full(E, 512, 512),
            full(E, 512, 512),
            full(E, 1, 512),
            full(K, 512),
            full(512, 1024),
            full(1, 1024),
            full(1024, LAT),
            full(1, LAT),
        ],
        out_specs=pl.BlockSpec((B, LAT), lambda i, be, st, en: (0, 0)),
    )
    return pl.pallas_call(
        _tc_body,
        grid_spec=grid_spec,
        out_shape=jax.ShapeDtypeStruct((B, LAT), jnp.float32),
    )(be, st, en, ts, W1, b12, W2, W3C1, b3C1, C1g, C2, CB2, C3, CB3)


def kernel(x, cats, W1, b1, W2, b2, W3, b3, CW1, CB1, CW2, CB2, CW3, CB3):
    cg = jnp.concatenate(
        [jnp.transpose(x[:, GEO:, :], (0, 2, 1)).reshape(N, SHAPE),
         jnp.transpose(x[:, :GEO, :], (0, 2, 1)).reshape(N, GEO),
         jnp.zeros((N, TW - SHAPE - GEO), jnp.float32)], axis=1)

    cf = cats.reshape(-1).astype(jnp.int32)
    oh = (cf[:, None] == jnp.arange(E, dtype=jnp.int32)).astype(jnp.int32)
    cum = jnp.cumsum(oh, axis=0)
    rank = jnp.sum((cum - oh) * oh, axis=1)
    counts = cum[-1]
    padded = ((counts + K - 1) // K) * K
    ends = jnp.cumsum(padded)
    off = ends - padded
    dest = jnp.sum(oh * off[None, :], axis=1) + rank

    ceb = jnp.sum(oh.reshape(B, P, E), axis=1)
    seb = jnp.cumsum(ceb, axis=0) - ceb

    bstart = jnp.arange(NBLK, dtype=jnp.int32) * K
    be = jnp.minimum(
        jnp.sum((bstart[:, None] >= ends[None, :]).astype(jnp.int32), axis=1),
        E - 1).astype(jnp.int32)
    beoh = (be[:, None] == jnp.arange(E, dtype=jnp.int32)).astype(jnp.int32)
    r0 = bstart - jnp.sum(beoh * off[None, :], axis=1)
    s_sel = jnp.sum(beoh[:, None, :] * seb[None, :, :], axis=2)
    c_sel = jnp.sum(beoh[:, None, :] * ceb[None, :, :], axis=2)
    st = jnp.clip(s_sel - r0[:, None], 0, K).astype(jnp.int32)
    en = jnp.clip(s_sel + c_sel - r0[:, None], 0, K).astype(jnp.int32)

    ts = _sc_scatter(cg, dest.reshape(NW, NCH, CH))

    bf16 = jnp.bfloat16
    C1e = CW1[GEO:]
    W3C1 = jnp.einsum('eij,jk->eik', W3, C1e)
    b3C1 = (b3 @ C1e + CB1[None, :]).reshape(E, 1, 512)
    C1g = jnp.concatenate(
        [CW1[:GEO], jnp.zeros((K - GEO, 512), jnp.float32)], axis=0)
    b12 = jnp.stack([b1, b2], axis=1)
    out = _tc_call(
        be, st, en, ts,
        W1.astype(bf16), b12, W2.astype(bf16),
        W3C1.astype(bf16), b3C1,
        C1g.astype(bf16), CW2.astype(bf16), CB2.reshape(1, 1024),
        CW3.astype(bf16), CB3.reshape(1, LAT))
    return out


# speedup vs baseline: 1.2029x; 1.1131x over previous
import jax
import jax.numpy as jnp
from jax import lax
from jax.experimental import pallas as pl
from jax.experimental.pallas import tpu as pltpu
from jax.experimental.pallas import tpu_sc as plsc

B, P = 4, 2048
N = B * P
SHAPE = 256
GEO = 4
E = 8
LAT = 1024
K = 256
NBLK = (N + E * (K - 1) + K - 1) // K
NPAD = NBLK * K
TW = 384

NEG = -3e38

NW = 32
RPW = N // NW
NCH = 2
CH = RPW // NCH


def _sc_scatter_body(cg_hbm, dest_hbm, out_hbm, idx_v, rows_v, sem_i, sem_o):
    wid = lax.axis_index("s") * 2 + lax.axis_index("c")
    pltpu.sync_copy(dest_hbm.at[wid], idx_v)
    loads = [pltpu.async_copy(cg_hbm.at[pl.ds(wid * RPW + j * CH, CH)],
                              rows_v.at[j], sem_i)
             for j in range(NCH)]
    stores = []
    for j in range(NCH):
        loads[j].wait()
        stores.append(
            pltpu.async_copy(rows_v.at[j], out_hbm.at[idx_v.at[j]], sem_o))
    for c in stores:
        c.wait()


def _sc_scatter(cg, dest3):
    mesh = plsc.VectorSubcoreMesh(core_axis_name="c", subcore_axis_name="s")
    return pl.kernel(
        _sc_scatter_body,
        mesh=mesh,
        out_type=jax.ShapeDtypeStruct((NPAD, TW), jnp.float32),
        scratch_types=[
            pltpu.VMEM((NCH, CH), jnp.int32),
            pltpu.VMEM((NCH, CH, TW), jnp.float32),
            pltpu.SemaphoreType.DMA,
            pltpu.SemaphoreType.DMA,
        ],
    )(cg, dest3)


def _tc_body(be_ref, st_ref, en_ref, ts_ref,
             W1_ref, b12_ref, W2_ref, W3C1_ref, b3C1_ref,
             C1g_ref, C2_ref, CB2_ref, C3_ref, CB3_ref,
             out_ref):
    i = pl.program_id(0)
    e = be_ref[i]
    bf16, f32 = jnp.bfloat16, jnp.float32

    @pl.when(i == 0)
    def _():
        out_ref[...] = jnp.full((B, LAT), NEG, dtype=jnp.float32)

    used = ((en_ref[i, 0] > st_ref[i, 0]) | (en_ref[i, 1] > st_ref[i, 1])
            | (en_ref[i, 2] > st_ref[i, 2]) | (en_ref[i, 3] > st_ref[i, 3]))

    @pl.when(used)
    def _():
        codes = ts_ref[:, :SHAPE].astype(bf16)
        bg = ts_ref[:, SHAPE:].astype(bf16)

        h = jnp.dot(codes, W1_ref[e],
                    preferred_element_type=f32) + b12_ref[e, :1]
        h = jnp.maximum(h, 0.0).astype(bf16)
        h = jnp.dot(h, W2_ref[e], preferred_element_type=f32) + b12_ref[e, 1:]
        h = jnp.maximum(h, 0.0).astype(bf16)

        t = (jnp.dot(h, W3C1_ref[e], preferred_element_type=f32)
             + jnp.dot(bg, C1g_ref[...], preferred_element_type=f32)
             + b3C1_ref[e])
        t = jnp.maximum(t, 0.0).astype(bf16)
        t = jnp.dot(t, C2_ref[...], preferred_element_type=f32) + CB2_ref[...]
        t = jnp.maximum(t, 0.0).astype(bf16)
        t = jnp.dot(t, C3_ref[...], preferred_element_type=f32) + CB3_ref[...]

        iota = lax.broadcasted_iota(jnp.int32, (K, 1), 0)
        mx = [jnp.max(jnp.where((iota >= st_ref[i, b])
                                & (iota < en_ref[i, b]), t, NEG), axis=0)
              for b in range(B)]
        res = jnp.stack(mx)
        out_ref[...] = jnp.maximum(out_ref[...], res)


def _tc_call(be, st, en, ts, W1, b12, W2, W3C1, b3C1, C1g, C2, CB2, C3, CB3):
    full = lambda *shape: pl.BlockSpec(shape, lambda i, be, st, en:
                                       (0,) * len(shape))
    grid_spec = pltpu.PrefetchScalarGridSpec(
        num_scalar_prefetch=3,
        grid=(NBLK,),
        in_specs=[
            pl.BlockSpec((K, TW), lambda i, be, st, en: (i, 0)),
            full(E, SHAPE, 512),
            full(E, 2, 512),
            full(E, 512, 512),
            full(E, 512, 512),
            full(E, 1, 512),
            full(TW - SHAPE, 512),
            full(512, 1024),
            full(1, 1024),
            full(1024, LAT),
            full(1, LAT),
        ],
        out_specs=pl.BlockSpec((B, LAT), lambda i, be, st, en: (0, 0)),
    )
    return pl.pallas_call(
        _tc_body,
        grid_spec=grid_spec,
        out_shape=jax.ShapeDtypeStruct((B, LAT), jnp.float32),
    )(be, st, en, ts, W1, b12, W2, W3C1, b3C1, C1g, C2, CB2, C3, CB3)


def kernel(x, cats, W1, b1, W2, b2, W3, b3, CW1, CB1, CW2, CB2, CW3, CB3):
    cg = jnp.concatenate(
        [jnp.transpose(x[:, GEO:, :], (0, 2, 1)).reshape(N, SHAPE),
         jnp.transpose(x[:, :GEO, :], (0, 2, 1)).reshape(N, GEO),
         jnp.zeros((N, TW - SHAPE - GEO), jnp.float32)], axis=1)

    cf = cats.reshape(-1).astype(jnp.int32)
    oh = (cf[:, None] == jnp.arange(E, dtype=jnp.int32)).astype(jnp.int32)
    cum = jnp.cumsum(oh, axis=0)
    rank = jnp.sum((cum - oh) * oh, axis=1)
    counts = cum[-1]
    padded = ((counts + K - 1) // K) * K
    ends = jnp.cumsum(padded)
    off = ends - padded
    dest = jnp.sum(oh * off[None, :], axis=1) + rank

    ceb = jnp.sum(oh.reshape(B, P, E), axis=1)
    seb = jnp.cumsum(ceb, axis=0) - ceb

    bstart = jnp.arange(NBLK, dtype=jnp.int32) * K
    be = jnp.minimum(
        jnp.sum((bstart[:, None] >= ends[None, :]).astype(jnp.int32), axis=1),
        E - 1).astype(jnp.int32)
    beoh = (be[:, None] == jnp.arange(E, dtype=jnp.int32)).astype(jnp.int32)
    r0 = bstart - jnp.sum(beoh * off[None, :], axis=1)
    s_sel = jnp.sum(beoh[:, None, :] * seb[None, :, :], axis=2)
    c_sel = jnp.sum(beoh[:, None, :] * ceb[None, :, :], axis=2)
    st = jnp.clip(s_sel - r0[:, None], 0, K).astype(jnp.int32)
    en = jnp.clip(s_sel + c_sel - r0[:, None], 0, K).astype(jnp.int32)

    ts = _sc_scatter(cg, dest.reshape(NW, NCH, CH))

    bf16 = jnp.bfloat16
    C1e = CW1[GEO:]
    W3C1 = jnp.einsum('eij,jk->eik', W3, C1e)
    b3C1 = (b3 @ C1e + CB1[None, :]).reshape(E, 1, 512)
    C1g = jnp.concatenate(
        [CW1[:GEO], jnp.zeros((TW - SHAPE - GEO, 512), jnp.float32)], axis=0)
    b12 = jnp.stack([b1, b2], axis=1)
    out = _tc_call(
        be, st, en, ts,
        W1.astype(bf16), b12, W2.astype(bf16),
        W3C1.astype(bf16), b3C1,
        C1g.astype(bf16), CW2.astype(bf16), CB2.reshape(1, 1024),
        CW3.astype(bf16), CB3.reshape(1, LAT))
    return out


# speedup vs baseline: 1.2472x; 1.0368x over previous
import jax
import jax.numpy as jnp
from jax import lax
from jax.experimental import pallas as pl
from jax.experimental.pallas import tpu as pltpu
from jax.experimental.pallas import tpu_sc as plsc

B, P = 4, 2048
N = B * P
SHAPE = 256
GEO = 4
E = 8
LAT = 1024
K = 512
NBLK = (N + E * (K - 1) + K - 1) // K
NPAD = NBLK * K
TW = 384

NEG = -3e38

NW = 32
RPW = N // NW
NCH = 2
CH = RPW // NCH


def _sc_scatter_body(cg_hbm, dest_hbm, out_hbm, idx_v, rows_v, sem_i, sem_o):
    wid = lax.axis_index("s") * 2 + lax.axis_index("c")
    pltpu.sync_copy(dest_hbm.at[wid], idx_v)
    loads = [pltpu.async_copy(cg_hbm.at[pl.ds(wid * RPW + j * CH, CH)],
                              rows_v.at[j], sem_i)
             for j in range(NCH)]
    stores = []
    for j in range(NCH):
        loads[j].wait()
        stores.append(
            pltpu.async_copy(rows_v.at[j], out_hbm.at[idx_v.at[j]], sem_o))
    for c in stores:
        c.wait()


def _sc_scatter(cg, dest3):
    mesh = plsc.VectorSubcoreMesh(core_axis_name="c", subcore_axis_name="s")
    return pl.kernel(
        _sc_scatter_body,
        mesh=mesh,
        out_type=jax.ShapeDtypeStruct((NPAD, TW), jnp.float32),
        scratch_types=[
            pltpu.VMEM((NCH, CH), jnp.int32),
            pltpu.VMEM((NCH, CH, TW), jnp.float32),
            pltpu.SemaphoreType.DMA,
            pltpu.SemaphoreType.DMA,
        ],
    )(cg, dest3)


def _tc_body(be_ref, st_ref, en_ref, ts_ref,
             W1_ref, b12_ref, W2_ref, W3C1_ref, b3C1_ref,
             C1g_ref, C2_ref, CB2_ref, C3_ref, CB3_ref,
             out_ref):
    i = pl.program_id(0)
    e = be_ref[i]
    bf16, f32 = jnp.bfloat16, jnp.float32

    @pl.when(i == 0)
    def _():
        out_ref[...] = jnp.full((B, LAT), NEG, dtype=jnp.float32)

    used = ((en_ref[i, 0] > st_ref[i, 0]) | (en_ref[i, 1] > st_ref[i, 1])
            | (en_ref[i, 2] > st_ref[i, 2]) | (en_ref[i, 3] > st_ref[i, 3]))

    @pl.when(used)
    def _():
        codes = ts_ref[:, :SHAPE].astype(bf16)
        bg = ts_ref[:, SHAPE:].astype(bf16)

        h = jnp.dot(codes, W1_ref[e],
                    preferred_element_type=f32) + b12_ref[e, :1]
        h = jnp.maximum(h, 0.0).astype(bf16)
        h = jnp.dot(h, W2_ref[e], preferred_element_type=f32) + b12_ref[e, 1:]
        h = jnp.maximum(h, 0.0).astype(bf16)

        t = (jnp.dot(h, W3C1_ref[e], preferred_element_type=f32)
             + jnp.dot(bg, C1g_ref[...], preferred_element_type=f32)
             + b3C1_ref[e])
        t = jnp.maximum(t, 0.0).astype(bf16)
        t = jnp.dot(t, C2_ref[...], preferred_element_type=f32) + CB2_ref[...]
        t = jnp.maximum(t, 0.0).astype(bf16)
        t = jnp.dot(t, C3_ref[...], preferred_element_type=f32) + CB3_ref[...]

        iota = lax.broadcasted_iota(jnp.int32, (K, 1), 0)
        mx = [jnp.max(jnp.where((iota >= st_ref[i, b])
                                & (iota < en_ref[i, b]), t, NEG), axis=0)
              for b in range(B)]
        res = jnp.stack(mx)
        out_ref[...] = jnp.maximum(out_ref[...], res)


def _tc_call(be, st, en, ts, W1, b12, W2, W3C1, b3C1, C1g, C2, CB2, C3, CB3):
    full = lambda *shape: pl.BlockSpec(shape, lambda i, be, st, en:
                                       (0,) * len(shape))
    grid_spec = pltpu.PrefetchScalarGridSpec(
        num_scalar_prefetch=3,
        grid=(NBLK,),
        in_specs=[
            pl.BlockSpec((K, TW), lambda i, be, st, en: (i, 0)),
            full(E, SHAPE, 512),
            full(E, 2, 512),
            full(E, 512, 512),
            full(E, 512, 512),
            full(E, 1, 512),
            full(TW - SHAPE, 512),
            full(512, 1024),
            full(1, 1024),
            full(1024, LAT),
            full(1, LAT),
        ],
        out_specs=pl.BlockSpec((B, LAT), lambda i, be, st, en: (0, 0)),
    )
    return pl.pallas_call(
        _tc_body,
        grid_spec=grid_spec,
        out_shape=jax.ShapeDtypeStruct((B, LAT), jnp.float32),
    )(be, st, en, ts, W1, b12, W2, W3C1, b3C1, C1g, C2, CB2, C3, CB3)


def kernel(x, cats, W1, b1, W2, b2, W3, b3, CW1, CB1, CW2, CB2, CW3, CB3):
    cg = jnp.concatenate(
        [jnp.transpose(x[:, GEO:, :], (0, 2, 1)).reshape(N, SHAPE),
         jnp.transpose(x[:, :GEO, :], (0, 2, 1)).reshape(N, GEO),
         jnp.zeros((N, TW - SHAPE - GEO), jnp.float32)], axis=1)

    cf = cats.reshape(-1).astype(jnp.int32)
    oh = (cf[:, None] == jnp.arange(E, dtype=jnp.int32)).astype(jnp.int32)
    cum = jnp.cumsum(oh, axis=0)
    rank = jnp.sum((cum - oh) * oh, axis=1)
    counts = cum[-1]
    padded = ((counts + K - 1) // K) * K
    ends = jnp.cumsum(padded)
    off = ends - padded
    dest = jnp.sum(oh * off[None, :], axis=1) + rank

    ceb = jnp.sum(oh.reshape(B, P, E), axis=1)
    seb = jnp.cumsum(ceb, axis=0) - ceb

    bstart = jnp.arange(NBLK, dtype=jnp.int32) * K
    be = jnp.minimum(
        jnp.sum((bstart[:, None] >= ends[None, :]).astype(jnp.int32), axis=1),
        E - 1).astype(jnp.int32)
    beoh = (be[:, None] == jnp.arange(E, dtype=jnp.int32)).astype(jnp.int32)
    r0 = bstart - jnp.sum(beoh * off[None, :], axis=1)
    s_sel = jnp.sum(beoh[:, None, :] * seb[None, :, :], axis=2)
    c_sel = jnp.sum(beoh[:, None, :] * ceb[None, :, :], axis=2)
    st = jnp.clip(s_sel - r0[:, None], 0, K).astype(jnp.int32)
    en = jnp.clip(s_sel + c_sel - r0[:, None], 0, K).astype(jnp.int32)

    ts = _sc_scatter(cg, dest.reshape(NW, NCH, CH))

    bf16 = jnp.bfloat16
    C1e = CW1[GEO:]
    W3C1 = jnp.einsum('eij,jk->eik', W3, C1e)
    b3C1 = (b3 @ C1e + CB1[None, :]).reshape(E, 1, 512)
    C1g = jnp.concatenate(
        [CW1[:GEO], jnp.zeros((TW - SHAPE - GEO, 512), jnp.float32)], axis=0)
    b12 = jnp.stack([b1, b2], axis=1)
    out = _tc_call(
        be, st, en, ts,
        W1.astype(bf16), b12, W2.astype(bf16),
        W3C1.astype(bf16), b3C1,
        C1g.astype(bf16), CW2.astype(bf16), CB2.reshape(1, 1024),
        CW3.astype(bf16), CB3.reshape(1, LAT))
    return out


# speedup vs baseline: 1.3426x; 1.0765x over previous
import jax
import jax.numpy as jnp
from jax import lax
from jax.experimental import pallas as pl
from jax.experimental.pallas import tpu as pltpu
from jax.experimental.pallas import tpu_sc as plsc

B, P = 4, 2048
N = B * P
SHAPE = 256
GEO = 4
E = 8
LAT = 1024
K = 512
NBLK = (N + E * (K - 1) + K - 1) // K
NPAD = NBLK * K
TW = 384

NEG = -3e38

NW = 32
RPW = N // NW
NCH = 2
CH = RPW // NCH


def _sc_scatter_body(cg_hbm, dest_hbm, out_hbm, idx_v, rows_v, sem_i, sem_o):
    wid = lax.axis_index("s") * 2 + lax.axis_index("c")
    pltpu.sync_copy(dest_hbm.at[wid], idx_v)
    loads = [pltpu.async_copy(cg_hbm.at[pl.ds(wid * RPW + j * CH, CH)],
                              rows_v.at[j], sem_i)
             for j in range(NCH)]
    stores = []
    for j in range(NCH):
        loads[j].wait()
        stores.append(
            pltpu.async_copy(rows_v.at[j], out_hbm.at[idx_v.at[j]], sem_o))
    for c in stores:
        c.wait()


def _sc_scatter(cg, dest3):
    mesh = plsc.VectorSubcoreMesh(core_axis_name="c", subcore_axis_name="s")
    return pl.kernel(
        _sc_scatter_body,
        mesh=mesh,
        out_type=jax.ShapeDtypeStruct((NPAD, TW), jnp.float32),
        scratch_types=[
            pltpu.VMEM((NCH, CH), jnp.int32),
            pltpu.VMEM((NCH, CH, TW), jnp.float32),
            pltpu.SemaphoreType.DMA,
            pltpu.SemaphoreType.DMA,
        ],
    )(cg, dest3)


def _tc_body(be_ref, st_ref, en_ref, ts_ref,
             W1_ref, b12_ref, W2_ref, W3C1_ref, b3C1_ref,
             C1g_ref, C2_ref, CB2_ref, C3_ref, CB3_ref,
             out_ref):
    i = pl.program_id(0)
    e = be_ref[i]
    bf16, f32 = jnp.bfloat16, jnp.float32

    @pl.when(i == 0)
    def _():
        out_ref[...] = jnp.full((B, LAT), NEG, dtype=jnp.float32)

    used = ((en_ref[i, 0] > st_ref[i, 0]) | (en_ref[i, 1] > st_ref[i, 1])
            | (en_ref[i, 2] > st_ref[i, 2]) | (en_ref[i, 3] > st_ref[i, 3]))

    @pl.when(used)
    def _():
        codes = ts_ref[:, :SHAPE].astype(bf16)
        bg = ts_ref[:, SHAPE:].astype(bf16)

        h = jnp.dot(codes, W1_ref[e],
                    preferred_element_type=f32) + b12_ref[e, :1]
        h = jnp.maximum(h, 0.0).astype(bf16)
        h = jnp.dot(h, W2_ref[e], preferred_element_type=f32) + b12_ref[e, 1:]
        h = jnp.maximum(h, 0.0).astype(bf16)

        t = (jnp.dot(h, W3C1_ref[e], preferred_element_type=f32)
             + jnp.dot(bg, C1g_ref[...], preferred_element_type=f32)
             + b3C1_ref[e])
        t = jnp.maximum(t, 0.0).astype(bf16)
        t = jnp.dot(t, C2_ref[...], preferred_element_type=f32) + CB2_ref[...]
        t = jnp.maximum(t, 0.0).astype(bf16)
        t = jnp.dot(t, C3_ref[...], preferred_element_type=f32) + CB3_ref[...]

        iota = lax.broadcasted_iota(jnp.int32, (K, 1), 0)
        mx = [jnp.max(jnp.where((iota >= st_ref[i, b])
                                & (iota < en_ref[i, b]), t, NEG), axis=0)
              for b in range(B)]
        res = jnp.stack(mx)
        out_ref[...] = jnp.maximum(out_ref[...], res)


def _tc_call(be, st, en, ts, W1, b12, W2, W3C1, b3C1, C1g, C2, CB2, C3, CB3):
    full = lambda *shape: pl.BlockSpec(shape, lambda i, be, st, en:
                                       (0,) * len(shape))
    grid_spec = pltpu.PrefetchScalarGridSpec(
        num_scalar_prefetch=3,
        grid=(NBLK,),
        in_specs=[
            pl.BlockSpec((K, TW), lambda i, be, st, en: (i, 0)),
            full(E, SHAPE, 512),
            full(E, 2, 512),
            full(E, 512, 512),
            full(E, 512, 512),
            full(E, 1, 512),
            full(TW - SHAPE, 512),
            full(512, 1024),
            full(1, 1024),
            full(1024, LAT),
            full(1, LAT),
        ],
        out_specs=pl.BlockSpec((B, LAT), lambda i, be, st, en: (0, 0)),
    )
    return pl.pallas_call(
        _tc_body,
        grid_spec=grid_spec,
        out_shape=jax.ShapeDtypeStruct((B, LAT), jnp.float32),
    )(be, st, en, ts, W1, b12, W2, W3C1, b3C1, C1g, C2, CB2, C3, CB3)


def kernel(x, cats, W1, b1, W2, b2, W3, b3, CW1, CB1, CW2, CB2, CW3, CB3):
    cg = jnp.concatenate(
        [jnp.transpose(x[:, GEO:, :], (0, 2, 1)).reshape(N, SHAPE),
         jnp.transpose(x[:, :GEO, :], (0, 2, 1)).reshape(N, GEO),
         jnp.zeros((N, TW - SHAPE - GEO), jnp.float32)], axis=1)

    cf = cats.reshape(-1).astype(jnp.int32)
    ohf = (cf[:, None] == jnp.arange(E, dtype=jnp.int32)).astype(jnp.float32)
    CB = 128
    oh3 = ohf.reshape(N // CB, CB, E)
    tril = jnp.tril(jnp.ones((CB, CB), jnp.float32), k=-1)
    intra = jnp.einsum('lk,bke->ble', tril, oh3)
    tot = jnp.sum(oh3, axis=1)
    pref = jnp.cumsum(tot, axis=0) - tot
    rankf = jnp.sum((intra + pref[:, None, :]) * oh3,
                    axis=2).reshape(N)
    counts = jnp.sum(tot, axis=0).astype(jnp.int32)
    padded = ((counts + K - 1) // K) * K
    ends = jnp.cumsum(padded)
    off = ends - padded
    offf = off.astype(jnp.float32)
    dest = (jnp.sum(ohf * offf[None, :], axis=1)
            + rankf).astype(jnp.int32)

    ceb = jnp.sum(tot.reshape(B, P // CB, E), axis=1).astype(jnp.int32)
    seb = jnp.cumsum(ceb, axis=0) - ceb

    bstart = jnp.arange(NBLK, dtype=jnp.int32) * K
    be = jnp.minimum(
        jnp.sum((bstart[:, None] >= ends[None, :]).astype(jnp.int32), axis=1),
        E - 1).astype(jnp.int32)
    beoh = (be[:, None] == jnp.arange(E, dtype=jnp.int32)).astype(jnp.int32)
    r0 = bstart - jnp.sum(beoh * off[None, :], axis=1)
    s_sel = jnp.sum(beoh[:, None, :] * seb[None, :, :], axis=2)
    c_sel = jnp.sum(beoh[:, None, :] * ceb[None, :, :], axis=2)
    st = jnp.clip(s_sel - r0[:, None], 0, K).astype(jnp.int32)
    en = jnp.clip(s_sel + c_sel - r0[:, None], 0, K).astype(jnp.int32)

    ts = _sc_scatter(cg, dest.reshape(NW, NCH, CH))

    bf16 = jnp.bfloat16
    C1e = CW1[GEO:]
    W3C1 = jnp.einsum('eij,jk->eik', W3, C1e)
    b3C1 = (b3 @ C1e + CB1[None, :]).reshape(E, 1, 512)
    C1g = jnp.concatenate(
        [CW1[:GEO], jnp.zeros((TW - SHAPE - GEO, 512), jnp.float32)], axis=0)
    b12 = jnp.stack([b1, b2], axis=1)
    out = _tc_call(
        be, st, en, ts,
        W1.astype(bf16), b12, W2.astype(bf16),
        W3C1.astype(bf16), b3C1,
        C1g.astype(bf16), CW2.astype(bf16), CB2.reshape(1, 1024),
        CW3.astype(bf16), CB3.reshape(1, LAT))
    return out
